# baseline (speedup 1.0000x reference)
import functools

import jax
import jax.numpy as jnp
from jax import lax
from jax.experimental import pallas as pl
from jax.experimental.pallas import tpu as pltpu
from jax.experimental.pallas import tpu_sc as plsc

_NROW = 100000
_HROW = _NROW // 2
_EMBEDDING_DIM = 64
_BATCH = 16384
_MARGIN = 1.0

_NC = 2
_NS = 16
_NW = _NC * _NS
_BPW = _BATCH // _NW
_CHUNK = 64
_NCHUNK = _BPW // _CHUNK
_L = 16
_UNROLL = 4

_CROWS = 1000
_GRID = _HROW // _CROWS


def _pack_body(ea_ref, eb_ref, ra_ref, rb_ref, ew_ref, rw_ref):
    ew_ref[:, 0:_EMBEDDING_DIM] = ea_ref[...]
    ew_ref[:, _EMBEDDING_DIM:128] = eb_ref[...]
    rw_ref[:, 0:_EMBEDDING_DIM] = ra_ref[...]
    rw_ref[:, _EMBEDDING_DIM:128] = rb_ref[...]


def _tec_body(pos_hbm, neg_hbm, ew_hbm, rw_hbm, out_hbm,
              ih_v, ir_v, it_v, jh_v, jr_v, jt_v,
              qih_v, qir_v, qit_v, qjh_v, qjr_v, qjt_v,
              b0h, b0r, b0t, b0nh, b0nr, b0nt,
              b1h, b1r, b1t, b1nh, b1nr, b1nt,
              acc_v, sem0, sem1):
    wid = lax.axis_index("s") * _NC + lax.axis_index("c")
    base = wid * _BPW
    lanes = lax.iota(jnp.int32, _L)
    zero = jnp.zeros((_L,), jnp.float32)

    idx_bufs = (ih_v, ir_v, it_v, jh_v, jr_v, jt_v)
    row_bufs = (qih_v, qir_v, qit_v, qjh_v, qjr_v, qjt_v)

    pltpu.sync_copy(pos_hbm.at[pl.ds(base, _BPW)], ih_v)
    pltpu.sync_copy(pos_hbm.at[pl.ds(_BATCH + base, _BPW)], ir_v)
    pltpu.sync_copy(pos_hbm.at[pl.ds(2 * _BATCH + base, _BPW)], it_v)
    pltpu.sync_copy(neg_hbm.at[pl.ds(base, _BPW)], jh_v)
    pltpu.sync_copy(neg_hbm.at[pl.ds(_BATCH + base, _BPW)], jr_v)
    pltpu.sync_copy(neg_hbm.at[pl.ds(2 * _BATCH + base, _BPW)], jt_v)

    half = jnp.full((_L,), _HROW, jnp.int32)

    def mk_rows(k, _):
        s = pl.ds(k * _L, _L)
        for ib, qb in zip(idx_bufs, row_bufs):
            v = ib[s]
            qb[s] = v - jnp.where(v < half, 0, half)
        return 0

    lax.fori_loop(0, _BPW // _L, mk_rows, 0)

    bufsets = ((b0h, b0r, b0t, b0nh, b0nr, b0nt),
               (b1h, b1r, b1t, b1nh, b1nr, b1nt))
    sems = (sem0, sem1)

    def issue(g, bufs, sem):
        s = pl.ds(g * _CHUNK, _CHUNK)
        return [
            pltpu.async_copy(ew_hbm.at[qih_v.at[s]], bufs[0], sem),
            pltpu.async_copy(rw_hbm.at[qir_v.at[s]], bufs[1], sem),
            pltpu.async_copy(ew_hbm.at[qit_v.at[s]], bufs[2], sem),
            pltpu.async_copy(ew_hbm.at[qjh_v.at[s]], bufs[3], sem),
            pltpu.async_copy(rw_hbm.at[qjr_v.at[s]], bufs[4], sem),
            pltpu.async_copy(ew_hbm.at[qjt_v.at[s]], bufs[5], sem),
        ]

    def compute_chunk(g, bufs, loss_in):
        def group(j0, loss_c):
            rows = j0 * _L + lanes
            off = g * _CHUNK
            cbs = [
                jnp.where(
                    ib[pl.ds(off + j0 * _L, _L)] < half, 0, _EMBEDDING_DIM
                )
                for ib in idx_bufs
            ]

            def dstep(i, carry):
                accs = list(carry)
                d0 = i * _UNROLL
                for k in range(_UNROLL):
                    cols = jnp.bitwise_and(
                        d0 + k + lanes, _EMBEDDING_DIM - 1
                    )
                    vals = [
                        plsc.load_gather(b, [rows, cb + cols])
                        for b, cb in zip(bufs, cbs)
                    ]
                    accs[2 * k] += jnp.abs(vals[0] + vals[1] - vals[2])
                    accs[2 * k + 1] += jnp.abs(vals[3] + vals[4] - vals[5])
                return tuple(accs)

            accs = lax.fori_loop(
                0, _EMBEDDING_DIM // _UNROLL, dstep, (zero,) * (2 * _UNROLL)
            )
            pd = (accs[0] + accs[2]) + (accs[4] + accs[6])
            nd = (accs[1] + accs[3]) + (accs[5] + accs[7])
            return loss_c + jnp.maximum(pd - nd + _MARGIN, 0.0)

        return lax.fori_loop(0, _CHUNK // _L, group, loss_in)

    loss = zero
    pend = issue(0, bufsets[0], sems[0])
    for g in range(_NCHUNK):
        for cp in pend:
            cp.wait()
        cur = bufsets[g % 2]
        if g + 1 < _NCHUNK:
            pend = issue(g + 1, bufsets[(g + 1) % 2], sems[(g + 1) % 2])
        loss = compute_chunk(g, cur, loss)

    acc_v[...] = loss * (1.0 / _BATCH)
    pltpu.sync_copy(acc_v, out_hbm.at[wid])


@jax.jit
def kernel(positive_triples, negative_triples, entity_weight, relation_weight):
    pos = positive_triples.reshape(-1)
    neg = negative_triples.reshape(-1)

    ew, rw = pl.pallas_call(
        _pack_body,
        grid=(_GRID,),
        in_specs=[
            pl.BlockSpec((_CROWS, _EMBEDDING_DIM), lambda i: (i, 0)),
            pl.BlockSpec((_CROWS, _EMBEDDING_DIM), lambda i: (i + _GRID, 0)),
            pl.BlockSpec((_CROWS, _EMBEDDING_DIM), lambda i: (i, 0)),
            pl.BlockSpec((_CROWS, _EMBEDDING_DIM), lambda i: (i + _GRID, 0)),
        ],
        out_specs=[
            pl.BlockSpec((_CROWS, 128), lambda i: (i, 0)),
            pl.BlockSpec((_CROWS, 128), lambda i: (i, 0)),
        ],
        out_shape=[
            jax.ShapeDtypeStruct((_HROW, 128), jnp.float32),
            jax.ShapeDtypeStruct((_HROW, 128), jnp.float32),
        ],
    )(entity_weight, entity_weight, relation_weight, relation_weight)

    mesh = plsc.VectorSubcoreMesh(core_axis_name="c", subcore_axis_name="s")
    f = functools.partial(
        pl.kernel,
        mesh=mesh,
        compiler_params=pltpu.CompilerParams(
            needs_layout_passes=False, use_tc_tiling_on_sc=True
        ),
        out_type=jax.ShapeDtypeStruct((_NW, _L), jnp.float32),
        scratch_types=(
            [pltpu.VMEM((_BPW,), jnp.int32)] * 12
            + [pltpu.VMEM((_CHUNK, 128), jnp.float32)] * 12
            + [pltpu.VMEM((_L,), jnp.float32),
               pltpu.SemaphoreType.DMA, pltpu.SemaphoreType.DMA]
        ),
    )(_tec_body)
    partial = f(pos, neg, ew, rw)
    return jnp.sum(partial)


# speedup vs baseline: 1.2095x; 1.2095x over previous
import functools

import jax
import jax.numpy as jnp
from jax import lax
from jax.experimental import pallas as pl
from jax.experimental.pallas import tpu as pltpu
from jax.experimental.pallas import tpu_sc as plsc

_EMBEDDING_DIM = 64
_BATCH = 16384
_MARGIN = 1.0

_NC = 2
_NS = 16
_NW = _NC * _NS
_BPW = _BATCH // _NW
_CHUNK = 128
_NCHUNK = _BPW // _CHUNK
_L = 16
_UNROLL = 4


def _tec_body(pos_hbm, neg_hbm, ent_hbm, rel_hbm, out_hbm,
              ih_v, ir_v, it_v, jh_v, jr_v, jt_v,
              ph0, pr0, pt0, nh0, nr0, nt0,
              ph1, pr1, pt1, nh1, nr1, nt1,
              acc_v, sem0, sem1):
    wid = lax.axis_index("s") * _NC + lax.axis_index("c")
    base = wid * _BPW
    lanes = lax.iota(jnp.int32, _L)
    zero = jnp.zeros((_L,), jnp.float32)

    pltpu.sync_copy(pos_hbm.at[pl.ds(base, _BPW)], ih_v)
    pltpu.sync_copy(pos_hbm.at[pl.ds(_BATCH + base, _BPW)], ir_v)
    pltpu.sync_copy(pos_hbm.at[pl.ds(2 * _BATCH + base, _BPW)], it_v)
    pltpu.sync_copy(neg_hbm.at[pl.ds(base, _BPW)], jh_v)
    pltpu.sync_copy(neg_hbm.at[pl.ds(_BATCH + base, _BPW)], jr_v)
    pltpu.sync_copy(neg_hbm.at[pl.ds(2 * _BATCH + base, _BPW)], jt_v)

    bufsets = ((ph0, pr0, pt0, nh0, nr0, nt0), (ph1, pr1, pt1, nh1, nr1, nt1))
    sems = (sem0, sem1)

    def issue(g, bufs, sem):
        s = pl.ds(g * _CHUNK, _CHUNK)
        return [
            pltpu.async_copy(ent_hbm.at[ih_v.at[s]], bufs[0], sem),
            pltpu.async_copy(rel_hbm.at[ir_v.at[s]], bufs[1], sem),
            pltpu.async_copy(ent_hbm.at[it_v.at[s]], bufs[2], sem),
            pltpu.async_copy(ent_hbm.at[jh_v.at[s]], bufs[3], sem),
            pltpu.async_copy(rel_hbm.at[jr_v.at[s]], bufs[4], sem),
            pltpu.async_copy(ent_hbm.at[jt_v.at[s]], bufs[5], sem),
        ]

    def compute_chunk(bufs, loss_in):
        ph, pr, pt, nh, nr, nt = bufs

        def group(j0, loss_c):
            rows = j0 * _L + lanes

            def dstep(i, carry):
                accs = list(carry)
                d0 = i * _UNROLL
                for k in range(_UNROLL):
                    cols = jnp.bitwise_and(
                        d0 + k + lanes, _EMBEDDING_DIM - 1
                    )
                    hp = plsc.load_gather(ph, [rows, cols])
                    rp = plsc.load_gather(pr, [rows, cols])
                    tp = plsc.load_gather(pt, [rows, cols])
                    hn = plsc.load_gather(nh, [rows, cols])
                    rn = plsc.load_gather(nr, [rows, cols])
                    tn = plsc.load_gather(nt, [rows, cols])
                    accs[k] = accs[k] + jnp.abs(hp + rp - tp)
                    accs[_UNROLL + k] = accs[_UNROLL + k] + jnp.abs(hn + rn - tn)
                return tuple(accs)

            accs = lax.fori_loop(
                0, _EMBEDDING_DIM // _UNROLL, dstep, (zero,) * (2 * _UNROLL)
            )
            pd = (accs[0] + accs[1]) + (accs[2] + accs[3])
            nd = (accs[4] + accs[5]) + (accs[6] + accs[7])
            return loss_c + jnp.maximum(pd - nd + _MARGIN, 0.0)

        return lax.fori_loop(0, _CHUNK // _L, group, loss_in)

    loss = zero
    pend = issue(0, bufsets[0], sems[0])
    for g in range(_NCHUNK):
        for cp in pend:
            cp.wait()
        cur = bufsets[g % 2]
        if g + 1 < _NCHUNK:
            pend = issue(g + 1, bufsets[(g + 1) % 2], sems[(g + 1) % 2])
        loss = compute_chunk(cur, loss)

    acc_v[...] = loss * (1.0 / _BATCH)
    pltpu.sync_copy(acc_v, out_hbm.at[wid])


@jax.jit
def kernel(positive_triples, negative_triples, entity_weight, relation_weight):
    pos = positive_triples.reshape(-1)
    neg = negative_triples.reshape(-1)
    mesh = plsc.VectorSubcoreMesh(core_axis_name="c", subcore_axis_name="s")
    f = functools.partial(
        pl.kernel,
        mesh=mesh,
        compiler_params=pltpu.CompilerParams(
            needs_layout_passes=False, use_tc_tiling_on_sc=False
        ),
        out_type=jax.ShapeDtypeStruct((_NW, _L), jnp.float32),
        scratch_types=(
            [pltpu.VMEM((_BPW,), jnp.int32)] * 6
            + [pltpu.VMEM((_CHUNK, _EMBEDDING_DIM), jnp.float32)] * 12
            + [pltpu.VMEM((_L,), jnp.float32),
               pltpu.SemaphoreType.DMA, pltpu.SemaphoreType.DMA]
        ),
    )(_tec_body)
    partial = f(pos, neg, entity_weight, relation_weight)
    return jnp.sum(partial)
